# baseline (speedup 1.0000x reference)
import functools
import jax
import jax.numpy as jnp
from jax import lax
from jax.experimental import pallas as pl
from jax.experimental.pallas import tpu as pltpu
from jax.experimental.pallas import tpu_sc as plsc

_N = 10000
_E = 320000
_NP = 10240
_CH = 128
_NCHUNK = _E // _CH
_NT = (_NCHUNK + 31) // 32
_ROWS = _NP // 16
_W = 80

_f32 = jnp.float32


def _pre_node_body(x_ref, w1_ref, b1_ref, w2_ref, b2_ref, wd_ref, ws_ref,
                   h_ref, ad_ref, t_ref):
    h1 = jnp.maximum(jnp.dot(x_ref[...], w1_ref[...],
                             preferred_element_type=_f32) + b1_ref[...], 0.0)
    h = jnp.maximum(jnp.dot(h1, w2_ref[...],
                            preferred_element_type=_f32) + b2_ref[...], 0.0)
    h_ref[...] = h
    ad_ref[...] = jnp.dot(h, wd_ref[...], preferred_element_type=_f32)
    a_s = jnp.dot(h, ws_ref[...], preferred_element_type=_f32)
    t_ref[...] = jnp.concatenate([a_s, h], axis=1)


def _pre_node(x, w1, b1, w2, b2, wd, ws):
    return pl.pallas_call(
        _pre_node_body,
        out_shape=(
            jax.ShapeDtypeStruct((_N, 64), _f32),
            jax.ShapeDtypeStruct((_N, 64), _f32),
            jax.ShapeDtypeStruct((_N, 128), _f32),
        ),
    )(x, w1, b1, w2, b2, wd, ws)


def _pre_edge_body(ea_ref, we_ref, b1_ref, et_ref):
    ea = ea_ref[...]
    e0 = jnp.dot(ea[:, :16], we_ref[...],
                 preferred_element_type=_f32) + b1_ref[...]
    e1 = jnp.dot(ea[:, 16:], we_ref[...],
                 preferred_element_type=_f32) + b1_ref[...]
    et_ref[...] = jnp.concatenate([e0, e1], axis=1)


def _pre_edge(ea2, we, b1):
    blk = 4000
    return pl.pallas_call(
        _pre_edge_body,
        grid=(_E // 2 // blk,),
        in_specs=[
            pl.BlockSpec((blk, 32), lambda i: (i, 0)),
            pl.BlockSpec((16, 64), lambda i: (0, 0)),
            pl.BlockSpec((1, 64), lambda i: (0, 0)),
        ],
        out_specs=pl.BlockSpec((blk, 128), lambda i: (i, 0)),
        out_shape=jax.ShapeDtypeStruct((_E // 2, 128), _f32),
    )(ea2, we, b1)


def _post_body(h_ref, u_ref, mw_ref, mb_ref,
               ow1_ref, ob1_ref, ow2_ref, ob2_ref, out_ref):
    us = u_ref[0] + u_ref[1]
    den = us[:_N, 64:65] + 1e-16
    ctx = us[:_N, :64] / den
    h = h_ref[...]
    hm = jnp.maximum(jnp.dot(jnp.concatenate([h, ctx], axis=1), mw_ref[...],
                             preferred_element_type=_f32) + mb_ref[...], 0.0)
    o1 = jnp.maximum(jnp.dot(jnp.concatenate([h, hm], axis=1), ow1_ref[...],
                             preferred_element_type=_f32) + ob1_ref[...], 0.0)
    out_ref[...] = jnp.dot(o1, ow2_ref[...],
                           preferred_element_type=_f32) + ob2_ref[...]


def _post(h, u, mw, mb, ow1, ob1, ow2, ob2):
    return pl.pallas_call(
        _post_body,
        out_shape=jax.ShapeDtypeStruct((_N, 2), _f32),
    )(h, u, mw, mb, ow1, ob1, ow2, ob2)


@functools.cache
def _sc_edge_fn():
    mesh = plsc.VectorSubcoreMesh(core_axis_name="c", subcore_axis_name="s",
                                  num_cores=2, num_subcores=16)

    @functools.partial(
        pl.kernel,
        out_type=jax.ShapeDtypeStruct((2, _NP, _W), _f32),
        mesh=mesh,
        compiler_params=pltpu.CompilerParams(needs_layout_passes=False,
                                             use_tc_tiling_on_sc=False),
        scratch_types=[
            pltpu.VMEM_SHARED((_NP, _W), _f32),
            pltpu.VMEM((_CH,), jnp.int32),
            pltpu.VMEM((_CH,), jnp.int32),
            pltpu.VMEM((_CH,), jnp.int32),
            pltpu.VMEM((_CH,), jnp.int32),
            pltpu.VMEM((_CH, 64), _f32),
            pltpu.VMEM((_CH, 64), _f32),
            pltpu.VMEM((_CH, 128), _f32),
            pltpu.VMEM((_CH, 128), _f32),
            pltpu.VMEM((_CH * 64,), _f32),
            pltpu.VMEM((_CH * 64,), _f32),
            pltpu.VMEM((_CH, _W), _f32),
            pltpu.VMEM((80,), _f32),
            pltpu.SemaphoreType.DMA,
            pltpu.SemaphoreType.DMA,
            pltpu.SemaphoreType.DMA,
            pltpu.SemaphoreType.DMA,
        ],
    )
    def _sc_edge(dst_hbm, src_hbm, ad_hbm, t_hbm, et_hbm, w_hbm,
                 u_out, u_sh, dst0, dst1, src0, src1, ad0, ad1, tb0, tb1,
                 et0, et1, ehstage, wbuf, semi0, semi1, semg0, semg1):
        cid = lax.axis_index("c")
        sid = lax.axis_index("s")
        wid = sid * 2 + cid

        dstb = (dst0, dst1)
        srcb = (src0, src1)
        adb = (ad0, ad1)
        tb = (tb0, tb1)
        etb = (et0, et1)
        semi = (semi0, semi1)
        semg = (semg0, semg1)

        pltpu.sync_copy(w_hbm, wbuf)

        w2 = [wbuf[pl.ds(k * 16, 16)] for k in range(4)]
        b2 = wbuf[pl.ds(64, 16)][0]
        iota = lax.iota(jnp.int32, 16)
        onehot = jnp.where(iota == 0, 1.0, 0.0).astype(_f32)
        zero16 = jnp.zeros((16,), _f32)

        def zero_body(r, carry):
            for kk in range(_W // 16):
                ehstage[r, pl.ds(kk * 16, 16)] = zero16
            return carry

        lax.fori_loop(0, _CH, zero_body, 0)
        for s0 in range(0, _ROWS, _CH):
            pltpu.sync_copy(ehstage, u_sh.at[pl.ds(sid * _ROWS + s0, _CH)])
        plsc.subcore_barrier()

        def issue_idx(b, t):
            off = (wid + 32 * t) * _CH
            pltpu.async_copy(dst_hbm.at[pl.ds(off, _CH)], dstb[b], semi[b])
            pltpu.async_copy(src_hbm.at[pl.ds(off, _CH)], srcb[b], semi[b])

        def wait_idx(b):
            pltpu.make_async_copy(dst_hbm.at[pl.ds(0, _CH)], dstb[b],
                                  semi[b]).wait()
            pltpu.make_async_copy(src_hbm.at[pl.ds(0, _CH)], srcb[b],
                                  semi[b]).wait()

        def issue_gather(b, t):
            offe = (wid + 32 * t) * (_CH * 64)
            pltpu.async_copy(ad_hbm.at[dstb[b]], adb[b], semg[b])
            pltpu.async_copy(t_hbm.at[srcb[b]], tb[b], semg[b])
            pltpu.async_copy(et_hbm.at[pl.ds(offe, _CH * 64)], etb[b],
                             semg[b])

        def wait_gather(b):
            pltpu.make_async_copy(ad_hbm.at[pl.ds(0, _CH)], adb[b],
                                  semg[b]).wait()
            pltpu.make_async_copy(t_hbm.at[pl.ds(0, _CH)], tb[b],
                                  semg[b]).wait()
            pltpu.make_async_copy(et_hbm.at[pl.ds(0, _CH * 64)], etb[b],
                                  semg[b]).wait()

        def compute(b):
            adr = adb[b]
            tr = tb[b]
            etr = etb[b]

            def group_body(g, carry2):
                base = g * 16
                sv = jnp.zeros((16,), _f32)
                for j in range(16):
                    r = base + j
                    acc = None
                    for kk in range(4):
                        a = adr[r, pl.ds(kk * 16, 16)]
                        bb = tr[r, pl.ds(kk * 16, 16)]
                        cc = etr[pl.ds(r * 64 + kk * 16, 16)]
                        t = jnp.maximum(a + bb + cc, 0.0)
                        p = t * w2[kk]
                        acc = p if acc is None else acc + p
                    sj = jnp.sum(acc)
                    sv = jnp.where(iota == j, sj, sv)
                ev = jnp.exp(sv + b2)
                for j in range(16):
                    r = base + j
                    es = ev[j]
                    for kk in range(4):
                        hrow = tr[r, pl.ds(64 + kk * 16, 16)]
                        ehstage[r, pl.ds(kk * 16, 16)] = hrow * es
                    ehstage[r, pl.ds(64, 16)] = es * onehot
                return carry2

            lax.fori_loop(0, _CH // 16, group_body, 0)

        issue_idx(0, 0)
        wait_idx(0)
        issue_gather(0, 0)
        issue_idx(1, 1)

        def pair_body(i, carry):
            for b in (0, 1):
                t = 2 * i + b

                @pl.when(wid + 32 * t < _NCHUNK)
                def _():
                    wait_gather(b)
                    nb = 1 - b

                    @pl.when(wid + 32 * (t + 1) < _NCHUNK)
                    def _():
                        wait_idx(nb)
                        issue_gather(nb, t + 1)

                    compute(b)
                    pltpu.sync_copy(ehstage, u_sh.at[dstb[b]], add=True)

                    @pl.when(wid + 32 * (t + 2) < _NCHUNK)
                    def _():
                        issue_idx(b, t + 2)

            return carry

        lax.fori_loop(0, (_NT + 1) // 2, pair_body, 0)

        plsc.subcore_barrier()
        pltpu.sync_copy(u_sh.at[pl.ds(sid * _ROWS, _ROWS)],
                        u_out.at[cid, pl.ds(sid * _ROWS, _ROWS)])

    return _sc_edge


def kernel(x, edge_index, edge_attr, enc_w1, enc_b1, enc_w2, enc_b2,
           att_w1, att_b1, att_w2, att_b2, mrg_w1, mrg_b1,
           out_w1, out_b1, out_w2, out_b2):
    wd = att_w1[:64]
    ws = att_w1[64:128]
    we = att_w1[128:]

    h, ad, tcat = _pre_node(x, enc_w1, enc_b1.reshape(1, 64),
                            enc_w2, enc_b2.reshape(1, 64), wd, ws)
    eterm = _pre_edge(edge_attr.reshape(_E // 2, 32), we,
                      att_b1.reshape(1, 64))

    wparams = jnp.concatenate([att_w2[:, 0], att_b2,
                               jnp.zeros((15,), _f32)])

    u = _sc_edge_fn()(edge_index[1], edge_index[0], ad, tcat,
                      eterm.reshape(_E * 64), wparams)

    return _post(h, u, mrg_w1, mrg_b1.reshape(1, 64),
                 out_w1, out_b1.reshape(1, 64), out_w2, out_b2.reshape(1, 2))


# speedup vs baseline: 1.0844x; 1.0844x over previous
import functools
import jax
import jax.numpy as jnp
from jax import lax
from jax.experimental import pallas as pl
from jax.experimental.pallas import tpu as pltpu
from jax.experimental.pallas import tpu_sc as plsc

_N = 10000
_E = 320000
_NP = 10240
_CH = 128
_NCHUNK = _E // _CH
_NT = (_NCHUNK + 31) // 32
_ROWS = _NP // 16
_W = 80

_f32 = jnp.float32


def _pre_node_body(x_ref, w1_ref, b1_ref, w2_ref, b2_ref, wd_ref, ws_ref,
                   h_ref, ad_ref, t_ref):
    h1 = jnp.maximum(jnp.dot(x_ref[...], w1_ref[...],
                             preferred_element_type=_f32) + b1_ref[...], 0.0)
    h = jnp.maximum(jnp.dot(h1, w2_ref[...],
                            preferred_element_type=_f32) + b2_ref[...], 0.0)
    h_ref[...] = h
    ad_ref[...] = jnp.dot(h, wd_ref[...], preferred_element_type=_f32)
    a_s = jnp.dot(h, ws_ref[...], preferred_element_type=_f32)
    t_ref[...] = jnp.concatenate([a_s, h], axis=1)


def _pre_node(x, w1, b1, w2, b2, wd, ws):
    return pl.pallas_call(
        _pre_node_body,
        out_shape=(
            jax.ShapeDtypeStruct((_N, 64), _f32),
            jax.ShapeDtypeStruct((_N, 64), _f32),
            jax.ShapeDtypeStruct((_N, 128), _f32),
        ),
    )(x, w1, b1, w2, b2, wd, ws)


def _pre_edge_body(ea_ref, we_ref, b1_ref, et_ref):
    ea = ea_ref[...]
    e0 = jnp.dot(ea[:, :16], we_ref[...],
                 preferred_element_type=_f32) + b1_ref[...]
    e1 = jnp.dot(ea[:, 16:], we_ref[...],
                 preferred_element_type=_f32) + b1_ref[...]
    et_ref[...] = jnp.concatenate([e0, e1], axis=1)


def _pre_edge(ea2, we, b1):
    blk = 4000
    return pl.pallas_call(
        _pre_edge_body,
        grid=(_E // 2 // blk,),
        in_specs=[
            pl.BlockSpec((blk, 32), lambda i: (i, 0)),
            pl.BlockSpec((16, 64), lambda i: (0, 0)),
            pl.BlockSpec((1, 64), lambda i: (0, 0)),
        ],
        out_specs=pl.BlockSpec((blk, 128), lambda i: (i, 0)),
        out_shape=jax.ShapeDtypeStruct((_E // 2, 128), _f32),
    )(ea2, we, b1)


def _post_body(h_ref, u_ref, mw_ref, mb_ref,
               ow1_ref, ob1_ref, ow2_ref, ob2_ref, out_ref):
    us = u_ref[0] + u_ref[1]
    den = us[:_N, 64:65] + 1e-16
    ctx = us[:_N, :64] / den
    h = h_ref[...]
    hm = jnp.maximum(jnp.dot(jnp.concatenate([h, ctx], axis=1), mw_ref[...],
                             preferred_element_type=_f32) + mb_ref[...], 0.0)
    o1 = jnp.maximum(jnp.dot(jnp.concatenate([h, hm], axis=1), ow1_ref[...],
                             preferred_element_type=_f32) + ob1_ref[...], 0.0)
    out_ref[...] = jnp.dot(o1, ow2_ref[...],
                           preferred_element_type=_f32) + ob2_ref[...]


def _post(h, u, mw, mb, ow1, ob1, ow2, ob2):
    return pl.pallas_call(
        _post_body,
        out_shape=jax.ShapeDtypeStruct((_N, 2), _f32),
    )(h, u, mw, mb, ow1, ob1, ow2, ob2)


@functools.cache
def _sc_edge_fn():
    mesh = plsc.VectorSubcoreMesh(core_axis_name="c", subcore_axis_name="s",
                                  num_cores=2, num_subcores=16)

    @functools.partial(
        pl.kernel,
        out_type=jax.ShapeDtypeStruct((2, _NP, _W), _f32),
        mesh=mesh,
        compiler_params=pltpu.CompilerParams(needs_layout_passes=False,
                                             use_tc_tiling_on_sc=False),
        scratch_types=[
            pltpu.VMEM_SHARED((_NP, _W), _f32),
            pltpu.VMEM((_CH,), jnp.int32),
            pltpu.VMEM((_CH,), jnp.int32),
            pltpu.VMEM((_CH,), jnp.int32),
            pltpu.VMEM((_CH,), jnp.int32),
            pltpu.VMEM((_CH, 64), _f32),
            pltpu.VMEM((_CH, 64), _f32),
            pltpu.VMEM((_CH, 128), _f32),
            pltpu.VMEM((_CH, 128), _f32),
            pltpu.VMEM((_CH * 64,), _f32),
            pltpu.VMEM((_CH * 64,), _f32),
            pltpu.VMEM((_CH, _W), _f32),
            pltpu.VMEM((80,), _f32),
            pltpu.SemaphoreType.DMA,
            pltpu.SemaphoreType.DMA,
            pltpu.SemaphoreType.DMA,
            pltpu.SemaphoreType.DMA,
        ],
    )
    def _sc_edge(dst_hbm, src_hbm, ad_hbm, t_hbm, et_hbm, w_hbm,
                 u_out, u_sh, dst0, dst1, src0, src1, ad0, ad1, tb0, tb1,
                 et0, et1, ehstage, wbuf, semi0, semi1, semg0, semg1):
        cid = lax.axis_index("c")
        sid = lax.axis_index("s")
        wid = sid * 2 + cid

        dstb = (dst0, dst1)
        srcb = (src0, src1)
        adb = (ad0, ad1)
        tb = (tb0, tb1)
        etb = (et0, et1)
        semi = (semi0, semi1)
        semg = (semg0, semg1)

        pltpu.sync_copy(w_hbm, wbuf)

        w2 = [wbuf[pl.ds(k * 16, 16)] for k in range(4)]
        b2 = wbuf[pl.ds(64, 16)][0]
        iota = lax.iota(jnp.int32, 16)
        onehot = jnp.where(iota == 0, 1.0, 0.0).astype(_f32)
        zero16 = jnp.zeros((16,), _f32)

        def zero_body(r, carry):
            for kk in range(_W // 16):
                ehstage[r, pl.ds(kk * 16, 16)] = zero16
            return carry

        lax.fori_loop(0, _CH, zero_body, 0)
        for s0 in range(0, _ROWS, _CH):
            pltpu.sync_copy(ehstage, u_sh.at[pl.ds(sid * _ROWS + s0, _CH)])
        plsc.subcore_barrier()

        def issue_idx(b, t):
            off = (wid + 32 * t) * _CH
            pltpu.async_copy(dst_hbm.at[pl.ds(off, _CH)], dstb[b], semi[b])
            pltpu.async_copy(src_hbm.at[pl.ds(off, _CH)], srcb[b], semi[b])

        def wait_idx(b):
            pltpu.make_async_copy(dst_hbm.at[pl.ds(0, _CH)], dstb[b],
                                  semi[b]).wait()
            pltpu.make_async_copy(src_hbm.at[pl.ds(0, _CH)], srcb[b],
                                  semi[b]).wait()

        def issue_gather(b, t):
            offe = (wid + 32 * t) * (_CH * 64)
            pltpu.async_copy(ad_hbm.at[dstb[b]], adb[b], semg[b])
            pltpu.async_copy(t_hbm.at[srcb[b]], tb[b], semg[b])
            pltpu.async_copy(et_hbm.at[pl.ds(offe, _CH * 64)], etb[b],
                             semg[b])

        def wait_gather(b):
            pltpu.make_async_copy(ad_hbm.at[pl.ds(0, _CH)], adb[b],
                                  semg[b]).wait()
            pltpu.make_async_copy(t_hbm.at[pl.ds(0, _CH)], tb[b],
                                  semg[b]).wait()
            pltpu.make_async_copy(et_hbm.at[pl.ds(0, _CH * 64)], etb[b],
                                  semg[b]).wait()

        def compute(b):
            adr = adb[b]
            tr = tb[b]
            etr = etb[b]

            @plsc.parallel_loop(0, _CH // 16, unroll=2)
            def group_body(g):
                base = g * 16
                sv = jnp.zeros((16,), _f32)
                for j in range(16):
                    r = base + j
                    acc = None
                    for kk in range(4):
                        a = adr[r, pl.ds(kk * 16, 16)]
                        bb = tr[r, pl.ds(kk * 16, 16)]
                        cc = etr[pl.ds(r * 64 + kk * 16, 16)]
                        t = jnp.maximum(a + bb + cc, 0.0)
                        p = t * w2[kk]
                        acc = p if acc is None else acc + p
                    sj = jnp.sum(acc)
                    sv = jnp.where(iota == j, sj, sv)
                ev = jnp.exp(sv + b2)
                for j in range(16):
                    r = base + j
                    es = ev[j]
                    for kk in range(4):
                        hrow = tr[r, pl.ds(64 + kk * 16, 16)]
                        ehstage[r, pl.ds(kk * 16, 16)] = hrow * es
                    ehstage[r, pl.ds(64, 16)] = es * onehot

        issue_idx(0, 0)
        wait_idx(0)
        issue_gather(0, 0)
        issue_idx(1, 1)

        def pair_body(i, carry):
            for b in (0, 1):
                t = 2 * i + b

                @pl.when(wid + 32 * t < _NCHUNK)
                def _():
                    wait_gather(b)
                    nb = 1 - b

                    @pl.when(wid + 32 * (t + 1) < _NCHUNK)
                    def _():
                        wait_idx(nb)
                        issue_gather(nb, t + 1)

                    compute(b)
                    pltpu.sync_copy(ehstage, u_sh.at[dstb[b]], add=True)

                    @pl.when(wid + 32 * (t + 2) < _NCHUNK)
                    def _():
                        issue_idx(b, t + 2)

            return carry

        lax.fori_loop(0, (_NT + 1) // 2, pair_body, 0)

        plsc.subcore_barrier()
        pltpu.sync_copy(u_sh.at[pl.ds(sid * _ROWS, _ROWS)],
                        u_out.at[cid, pl.ds(sid * _ROWS, _ROWS)])

    return _sc_edge


def kernel(x, edge_index, edge_attr, enc_w1, enc_b1, enc_w2, enc_b2,
           att_w1, att_b1, att_w2, att_b2, mrg_w1, mrg_b1,
           out_w1, out_b1, out_w2, out_b2):
    wd = att_w1[:64]
    ws = att_w1[64:128]
    we = att_w1[128:]

    h, ad, tcat = _pre_node(x, enc_w1, enc_b1.reshape(1, 64),
                            enc_w2, enc_b2.reshape(1, 64), wd, ws)
    eterm = _pre_edge(edge_attr.reshape(_E // 2, 32), we,
                      att_b1.reshape(1, 64))

    wparams = jnp.concatenate([att_w2[:, 0], att_b2,
                               jnp.zeros((15,), _f32)])

    u = _sc_edge_fn()(edge_index[1], edge_index[0], ad, tcat,
                      eterm.reshape(_E * 64), wparams)

    return _post(h, u, mrg_w1, mrg_b1.reshape(1, 64),
                 out_w1, out_b1.reshape(1, 64), out_w2, out_b2.reshape(1, 2))


# speedup vs baseline: 1.1573x; 1.0673x over previous
import functools
import jax
import jax.numpy as jnp
from jax import lax
from jax.experimental import pallas as pl
from jax.experimental.pallas import tpu as pltpu
from jax.experimental.pallas import tpu_sc as plsc

_N = 10000
_E = 320000
_NP = 10240
_CH = 128
_EH = _E // 2
_NCHUNK = _EH // _CH
_NT = (_NCHUNK + 31) // 32
_ROWS = _NP // 16
_W = 80

_f32 = jnp.float32


def _pre_node_body(x_ref, w1_ref, b1_ref, w2_ref, b2_ref, wd_ref, ws_ref,
                   h_ref, ad_ref, t_ref):
    h1 = jnp.maximum(jnp.dot(x_ref[...], w1_ref[...],
                             preferred_element_type=_f32) + b1_ref[...], 0.0)
    h = jnp.maximum(jnp.dot(h1, w2_ref[...],
                            preferred_element_type=_f32) + b2_ref[...], 0.0)
    h_ref[...] = h
    ad_ref[...] = jnp.dot(h, wd_ref[...], preferred_element_type=_f32)
    a_s = jnp.dot(h, ws_ref[...], preferred_element_type=_f32)
    t_ref[...] = jnp.concatenate([a_s, h], axis=1)


def _pre_node(x, w1, b1, w2, b2, wd, ws):
    return pl.pallas_call(
        _pre_node_body,
        out_shape=(
            jax.ShapeDtypeStruct((_N, 64), _f32),
            jax.ShapeDtypeStruct((_N, 64), _f32),
            jax.ShapeDtypeStruct((_N, 128), _f32),
        ),
    )(x, w1, b1, w2, b2, wd, ws)


def _pre_edge_body(ea_ref, we_ref, b1_ref, et_ref):
    ea = ea_ref[...]
    e0 = jnp.dot(ea[:, :16], we_ref[...],
                 preferred_element_type=_f32) + b1_ref[...]
    e1 = jnp.dot(ea[:, 16:], we_ref[...],
                 preferred_element_type=_f32) + b1_ref[...]
    et_ref[...] = jnp.concatenate([e0, e1], axis=1)


def _pre_edge(ea2, we, b1):
    blk = 4000
    nrow = _EH // 2
    return pl.pallas_call(
        _pre_edge_body,
        grid=(nrow // blk,),
        in_specs=[
            pl.BlockSpec((blk, 32), lambda i: (i, 0)),
            pl.BlockSpec((16, 64), lambda i: (0, 0)),
            pl.BlockSpec((1, 64), lambda i: (0, 0)),
        ],
        out_specs=pl.BlockSpec((blk, 128), lambda i: (i, 0)),
        out_shape=jax.ShapeDtypeStruct((nrow, 128), _f32),
    )(ea2, we, b1)


def _post_body(h_ref, u_ref, v_ref, mw_ref, mb_ref,
               ow1_ref, ob1_ref, ow2_ref, ob2_ref, out_ref):
    us = u_ref[0] + u_ref[1] + v_ref[0] + v_ref[1]
    den = us[:_N, 64:65] + 1e-16
    ctx = us[:_N, :64] / den
    h = h_ref[...]
    hm = jnp.maximum(jnp.dot(jnp.concatenate([h, ctx], axis=1), mw_ref[...],
                             preferred_element_type=_f32) + mb_ref[...], 0.0)
    o1 = jnp.maximum(jnp.dot(jnp.concatenate([h, hm], axis=1), ow1_ref[...],
                             preferred_element_type=_f32) + ob1_ref[...], 0.0)
    out_ref[...] = jnp.dot(o1, ow2_ref[...],
                           preferred_element_type=_f32) + ob2_ref[...]


def _post(h, u, v, mw, mb, ow1, ob1, ow2, ob2):
    return pl.pallas_call(
        _post_body,
        out_shape=jax.ShapeDtypeStruct((_N, 2), _f32),
    )(h, u, v, mw, mb, ow1, ob1, ow2, ob2)


@functools.cache
def _sc_edge_fn():
    mesh = plsc.VectorSubcoreMesh(core_axis_name="c", subcore_axis_name="s",
                                  num_cores=2, num_subcores=16)

    @functools.partial(
        pl.kernel,
        out_type=jax.ShapeDtypeStruct((2, _NP, _W), _f32),
        mesh=mesh,
        compiler_params=pltpu.CompilerParams(needs_layout_passes=False,
                                             use_tc_tiling_on_sc=False),
        scratch_types=[
            pltpu.VMEM_SHARED((_NP, _W), _f32),
            pltpu.VMEM((_CH,), jnp.int32),
            pltpu.VMEM((_CH,), jnp.int32),
            pltpu.VMEM((_CH,), jnp.int32),
            pltpu.VMEM((_CH,), jnp.int32),
            pltpu.VMEM((_CH, 64), _f32),
            pltpu.VMEM((_CH, 64), _f32),
            pltpu.VMEM((_CH, 128), _f32),
            pltpu.VMEM((_CH, 128), _f32),
            pltpu.VMEM((_CH * 64,), _f32),
            pltpu.VMEM((_CH * 64,), _f32),
            pltpu.VMEM((_CH, _W), _f32),
            pltpu.VMEM((80,), _f32),
            pltpu.SemaphoreType.DMA,
            pltpu.SemaphoreType.DMA,
            pltpu.SemaphoreType.DMA,
            pltpu.SemaphoreType.DMA,
        ],
    )
    def _sc_edge(dst_hbm, src_hbm, ad_hbm, t_hbm, et_hbm, w_hbm,
                 u_out, u_sh, dst0, dst1, src0, src1, ad0, ad1, tb0, tb1,
                 et0, et1, ehstage, wbuf, semi0, semi1, semg0, semg1):
        cid = lax.axis_index("c")
        sid = lax.axis_index("s")
        wid = sid * 2 + cid

        dstb = (dst0, dst1)
        srcb = (src0, src1)
        adb = (ad0, ad1)
        tb = (tb0, tb1)
        etb = (et0, et1)
        semi = (semi0, semi1)
        semg = (semg0, semg1)

        pltpu.sync_copy(w_hbm, wbuf)

        w2 = [wbuf[pl.ds(k * 16, 16)] for k in range(4)]
        b2 = wbuf[pl.ds(64, 16)][0]
        iota = lax.iota(jnp.int32, 16)
        onehot = jnp.where(iota == 0, 1.0, 0.0).astype(_f32)
        zero16 = jnp.zeros((16,), _f32)

        def zero_body(r, carry):
            for kk in range(_W // 16):
                ehstage[r, pl.ds(kk * 16, 16)] = zero16
            return carry

        lax.fori_loop(0, _CH, zero_body, 0)
        for s0 in range(0, _ROWS, _CH):
            pltpu.sync_copy(ehstage, u_sh.at[pl.ds(sid * _ROWS + s0, _CH)])
        plsc.subcore_barrier()

        def issue_idx(b, t):
            off = (wid + 32 * t) * _CH
            pltpu.async_copy(dst_hbm.at[pl.ds(off, _CH)], dstb[b], semi[b])
            pltpu.async_copy(src_hbm.at[pl.ds(off, _CH)], srcb[b], semi[b])

        def wait_idx(b):
            pltpu.make_async_copy(dst_hbm.at[pl.ds(0, _CH)], dstb[b],
                                  semi[b]).wait()
            pltpu.make_async_copy(src_hbm.at[pl.ds(0, _CH)], srcb[b],
                                  semi[b]).wait()

        def issue_gather(b, t):
            offe = (wid + 32 * t) * (_CH * 64)
            pltpu.async_copy(ad_hbm.at[dstb[b]], adb[b], semg[b])
            pltpu.async_copy(t_hbm.at[srcb[b]], tb[b], semg[b])
            pltpu.async_copy(et_hbm.at[pl.ds(offe, _CH * 64)], etb[b],
                             semg[b])

        def wait_gather(b):
            pltpu.make_async_copy(ad_hbm.at[pl.ds(0, _CH)], adb[b],
                                  semg[b]).wait()
            pltpu.make_async_copy(t_hbm.at[pl.ds(0, _CH)], tb[b],
                                  semg[b]).wait()
            pltpu.make_async_copy(et_hbm.at[pl.ds(0, _CH * 64)], etb[b],
                                  semg[b]).wait()

        def compute(b):
            adr = adb[b]
            tr = tb[b]
            etr = etb[b]

            @plsc.parallel_loop(0, _CH // 16, unroll=2)
            def group_body(g):
                base = g * 16
                sv = jnp.zeros((16,), _f32)
                for j in range(16):
                    r = base + j
                    acc = None
                    for kk in range(4):
                        a = adr[r, pl.ds(kk * 16, 16)]
                        bb = tr[r, pl.ds(kk * 16, 16)]
                        cc = etr[pl.ds(r * 64 + kk * 16, 16)]
                        t = jnp.maximum(a + bb + cc, 0.0)
                        p = t * w2[kk]
                        acc = p if acc is None else acc + p
                    sj = jnp.sum(acc)
                    sv = jnp.where(iota == j, sj, sv)
                ev = jnp.exp(sv + b2)
                for j in range(16):
                    r = base + j
                    es = ev[j]
                    for kk in range(4):
                        hrow = tr[r, pl.ds(64 + kk * 16, 16)]
                        ehstage[r, pl.ds(kk * 16, 16)] = hrow * es
                    ehstage[r, pl.ds(64, 16)] = es * onehot

        issue_idx(0, 0)
        wait_idx(0)
        issue_gather(0, 0)
        issue_idx(1, 1)

        def pair_body(i, carry):
            for b in (0, 1):
                t = 2 * i + b

                @pl.when(wid + 32 * t < _NCHUNK)
                def _():
                    wait_gather(b)
                    nb = 1 - b

                    @pl.when(wid + 32 * (t + 1) < _NCHUNK)
                    def _():
                        wait_idx(nb)
                        issue_gather(nb, t + 1)

                    compute(b)
                    pltpu.sync_copy(ehstage, u_sh.at[dstb[b]], add=True)

                    @pl.when(wid + 32 * (t + 2) < _NCHUNK)
                    def _():
                        issue_idx(b, t + 2)

            return carry

        lax.fori_loop(0, (_NT + 1) // 2, pair_body, 0)

        plsc.subcore_barrier()
        pltpu.sync_copy(u_sh.at[pl.ds(sid * _ROWS, _ROWS)],
                        u_out.at[cid, pl.ds(sid * _ROWS, _ROWS)])

    return _sc_edge


def kernel(x, edge_index, edge_attr, enc_w1, enc_b1, enc_w2, enc_b2,
           att_w1, att_b1, att_w2, att_b2, mrg_w1, mrg_b1,
           out_w1, out_b1, out_w2, out_b2):
    wd = att_w1[:64]
    ws = att_w1[64:128]
    we = att_w1[128:]

    h, ad, tcat = _pre_node(x, enc_w1, enc_b1.reshape(1, 64),
                            enc_w2, enc_b2.reshape(1, 64), wd, ws)
    ea2 = edge_attr.reshape(_E // 2, 32)
    b1r = att_b1.reshape(1, 64)
    et_a = _pre_edge(ea2[:_EH // 2], we, b1r)
    et_b = _pre_edge(ea2[_EH // 2:], we, b1r)

    wparams = jnp.concatenate([att_w2[:, 0], att_b2,
                               jnp.zeros((15,), _f32)])

    dst = edge_index[1]
    src = edge_index[0]
    sc = _sc_edge_fn()
    u = sc(dst[:_EH], src[:_EH], ad, tcat, et_a.reshape(_EH * 64), wparams)
    v = sc(dst[_EH:], src[_EH:], ad, tcat, et_b.reshape(_EH * 64), wparams)

    return _post(h, u, v, mrg_w1, mrg_b1.reshape(1, 64),
                 out_w1, out_b1.reshape(1, 64), out_w2, out_b2.reshape(1, 2))
